# Initial kernel scaffold; baseline (speedup 1.0000x reference)
import functools

import jax
import jax.numpy as jnp
from jax import lax
from jax.experimental import pallas as pl
from jax.experimental.pallas import tpu as pltpu
from jax.experimental.pallas import tpu_sc as plsc

N = 10000
E = 320000
D = 128
G = 64

NC = 2
NS = 16
NW = NC * NS
EDGES_PER_W = E // NW
CHUNK = 80
NCHUNK = EDGES_PER_W // CHUNK
ROWS_PER_TILE = N // NS
NPAD = 10240
DEG_PER_TILE = NPAD // NS

_sc_mesh = plsc.VectorSubcoreMesh(core_axis_name="c", subcore_axis_name="s")


@functools.partial(
    pl.kernel,
    out_type=jax.ShapeDtypeStruct((NC, NPAD), jnp.float32),
    mesh=_sc_mesh,
    scratch_types=[
        pltpu.VMEM((CHUNK,), jnp.int32),
        pltpu.VMEM((CHUNK,), jnp.float32),
        pltpu.VMEM_SHARED((NPAD,), jnp.float32),
    ],
)
def _deg_sc(dst_hbm, ew_hbm, zeros_hbm, out_hbm, idx_v, ew_v, acc_sp):
    c = lax.axis_index("c")
    s = lax.axis_index("s")
    wid = s * NC + c
    tile_lo = s * DEG_PER_TILE
    pltpu.sync_copy(zeros_hbm.at[pl.ds(tile_lo, DEG_PER_TILE)],
                    acc_sp.at[pl.ds(tile_lo, DEG_PER_TILE)])
    plsc.subcore_barrier()
    base = wid * EDGES_PER_W

    def chunk_body(i, carry):
        off = pl.multiple_of(base + i * CHUNK, 8)
        pltpu.sync_copy(dst_hbm.at[pl.ds(off, CHUNK)], idx_v)
        pltpu.sync_copy(ew_hbm.at[pl.ds(off, CHUNK)], ew_v)
        pltpu.sync_copy(ew_v, acc_sp.at[idx_v], add=True)
        return carry

    lax.fori_loop(0, NCHUNK, chunk_body, 0)
    plsc.subcore_barrier()
    pltpu.sync_copy(acc_sp.at[pl.ds(tile_lo, DEG_PER_TILE)],
                    out_hbm.at[c, pl.ds(tile_lo, DEG_PER_TILE)])


@functools.partial(
    pl.kernel,
    out_type=jax.ShapeDtypeStruct((NC, N, D), jnp.float32),
    mesh=_sc_mesh,
    scratch_types=[
        pltpu.VMEM((CHUNK,), jnp.int32),
        pltpu.VMEM((CHUNK,), jnp.int32),
        pltpu.VMEM((CHUNK,), jnp.float32),
        pltpu.VMEM((CHUNK, D), jnp.float32),
        pltpu.VMEM_SHARED((N, D), jnp.float32),
        pltpu.SemaphoreType.DMA,
    ],
)
def _agg_sc(g_hbm, src_hbm, dst_hbm, ew_hbm, zrows_hbm, out_hbm,
            sidx_v, didx_v, ew_v, rows_v, acc_sp, sem):
    c = lax.axis_index("c")
    s = lax.axis_index("s")
    wid = s * NC + c
    row_lo = s * ROWS_PER_TILE
    pltpu.sync_copy(zrows_hbm.at[pl.ds(row_lo, ROWS_PER_TILE)],
                    acc_sp.at[pl.ds(row_lo, ROWS_PER_TILE)])
    plsc.subcore_barrier()
    base = wid * EDGES_PER_W

    def chunk_body(i, carry):
        off = pl.multiple_of(base + i * CHUNK, 8)
        pltpu.sync_copy(src_hbm.at[pl.ds(off, CHUNK)], sidx_v)
        pltpu.sync_copy(dst_hbm.at[pl.ds(off, CHUNK)], didx_v)
        pltpu.sync_copy(ew_hbm.at[pl.ds(off, CHUNK)], ew_v)
        pltpu.async_copy(g_hbm.at[sidx_v], rows_v, sem).wait()

        def scale_body(e, c2):
            w = ew_v[e]
            for j in range(D // 16):
                sl = pl.ds(j * 16, 16)
                rows_v[e, sl] = rows_v[e, sl] * w
            return c2

        lax.fori_loop(0, CHUNK, scale_body, 0)
        pltpu.sync_copy(rows_v, acc_sp.at[didx_v], add=True)
        return carry

    lax.fori_loop(0, NCHUNK, chunk_body, 0)
    plsc.subcore_barrier()
    pltpu.sync_copy(acc_sp.at[pl.ds(row_lo, ROWS_PER_TILE)],
                    out_hbm.at[c, pl.ds(row_lo, ROWS_PER_TILE)])


def _mm1_body(x_ref, w_ref, degp_ref, g_ref, dinv_ref):
    deg = degp_ref[:, 0:1] + degp_ref[:, 1:2] + 1.0
    dinv = lax.rsqrt(deg)
    h = jnp.dot(x_ref[...], w_ref[...], preferred_element_type=jnp.float32)
    g_ref[...] = h * dinv
    dinv_ref[...] = dinv


def _mid_body(accp_ref, g_ref, dinv_ref, b_ref, w_ref, o_ref):
    a = accp_ref[0] + accp_ref[1] + g_ref[...]
    h = jnp.maximum(a * dinv_ref[...] + b_ref[...], 0.0)
    h2 = jnp.dot(h, w_ref[...], preferred_element_type=jnp.float32)
    o_ref[...] = h2 * dinv_ref[...]


def _final_body(accp_ref, g_ref, dinv_ref, b_ref, batch_ref, wh_ref, bh_ref,
                o_ref):
    a = accp_ref[0] + accp_ref[1] + g_ref[...]
    h = jnp.maximum(a * dinv_ref[...] + b_ref[...], 0.0)
    gids = lax.broadcasted_iota(jnp.int32, (G, N), 0)
    m = (batch_ref[...] == gids).astype(jnp.float32)
    sums = jnp.dot(m, h, preferred_element_type=jnp.float32)
    cnts = jnp.sum(m, axis=1, keepdims=True)
    pooled = sums / jnp.maximum(cnts, 1.0)
    o_ref[...] = (jnp.dot(pooled, wh_ref[...],
                          preferred_element_type=jnp.float32) + bh_ref[...])


_mm1 = pl.pallas_call(
    _mm1_body,
    out_shape=(jax.ShapeDtypeStruct((N, D), jnp.float32),
               jax.ShapeDtypeStruct((N, 1), jnp.float32)),
)

_mid = pl.pallas_call(
    _mid_body,
    out_shape=jax.ShapeDtypeStruct((N, D), jnp.float32),
)

_final = pl.pallas_call(
    _final_body,
    out_shape=jax.ShapeDtypeStruct((G, 1), jnp.float32),
)


def kernel(x, edge_index, edge_weight, batch, W1, b1, W2, b2, Wh, bh):
    src = edge_index[0]
    dst = edge_index[1]
    zeros_deg = jnp.zeros((NPAD,), jnp.float32)
    zeros_rows = jnp.zeros((N, D), jnp.float32)

    degp = _deg_sc(dst, edge_weight, zeros_deg)
    degp_t = degp[:, :N].T

    g1, dinv = _mm1(x, W1, degp_t)
    acc1 = _agg_sc(g1, src, dst, edge_weight, zeros_rows)
    g2 = _mid(acc1, g1, dinv, b1.reshape(1, D), W2)
    acc2 = _agg_sc(g2, src, dst, edge_weight, zeros_rows)
    out = _final(acc2, g2, dinv, b2.reshape(1, D), batch.reshape(1, N),
                 Wh, bh.reshape(1, 1))
    return out


# speedup vs baseline: 9.8709x; 9.8709x over previous
import functools

import jax
import jax.numpy as jnp
from jax import lax
from jax.experimental import pallas as pl
from jax.experimental.pallas import tpu as pltpu
from jax.experimental.pallas import tpu_sc as plsc

N = 10000
E = 320000
D = 128
G = 64

NC = 2
NS = 16
NW = NC * NS
EDGES_PER_W = E // NW
CHUNK = 80
NCHUNK = EDGES_PER_W // CHUNK
NPAD = 10240
ROWS_PER_TILE = NPAD // NS
DEG_PER_TILE = NPAD // NS

_sc_mesh = plsc.VectorSubcoreMesh(core_axis_name="c", subcore_axis_name="s")


@functools.partial(
    pl.kernel,
    out_type=jax.ShapeDtypeStruct((NC, NPAD), jnp.float32),
    mesh=_sc_mesh,
    scratch_types=[
        pltpu.VMEM((CHUNK,), jnp.int32),
        pltpu.VMEM((CHUNK,), jnp.float32),
        pltpu.VMEM_SHARED((NPAD,), jnp.float32),
    ],
)
def _deg_sc(dst_hbm, ew_hbm, zeros_hbm, out_hbm, idx_v, ew_v, acc_sp):
    c = lax.axis_index("c")
    s = lax.axis_index("s")
    wid = s * NC + c
    tile_lo = s * DEG_PER_TILE
    pltpu.sync_copy(zeros_hbm.at[pl.ds(tile_lo, DEG_PER_TILE)],
                    acc_sp.at[pl.ds(tile_lo, DEG_PER_TILE)])
    plsc.subcore_barrier()
    base = wid * EDGES_PER_W

    def chunk_body(i, carry):
        off = pl.multiple_of(base + i * CHUNK, 8)
        pltpu.sync_copy(dst_hbm.at[pl.ds(off, CHUNK)], idx_v)
        pltpu.sync_copy(ew_hbm.at[pl.ds(off, CHUNK)], ew_v)
        pltpu.sync_copy(ew_v, acc_sp.at[idx_v], add=True)
        return carry

    lax.fori_loop(0, NCHUNK, chunk_body, 0)
    plsc.subcore_barrier()
    pltpu.sync_copy(acc_sp.at[pl.ds(tile_lo, DEG_PER_TILE)],
                    out_hbm.at[c, pl.ds(tile_lo, DEG_PER_TILE)])


@functools.partial(
    pl.kernel,
    out_type=jax.ShapeDtypeStruct((NC, NPAD, D), jnp.float32),
    mesh=_sc_mesh,
    scratch_types=[
        pltpu.VMEM((CHUNK,), jnp.int32),
        pltpu.VMEM((CHUNK,), jnp.int32),
        pltpu.VMEM((CHUNK,), jnp.float32),
        pltpu.VMEM((CHUNK, D), jnp.float32),
        pltpu.VMEM_SHARED((NPAD, D), jnp.float32),
        pltpu.SemaphoreType.DMA,
    ],
)
def _agg_sc(g_hbm, src_hbm, dst_hbm, ew_hbm, zrows_hbm, out_hbm,
            sidx_v, didx_v, ew_v, rows_v, acc_sp, sem):
    c = lax.axis_index("c")
    s = lax.axis_index("s")
    wid = s * NC + c
    row_lo = s * ROWS_PER_TILE
    pltpu.sync_copy(zrows_hbm.at[pl.ds(row_lo, ROWS_PER_TILE)],
                    acc_sp.at[pl.ds(row_lo, ROWS_PER_TILE)])
    plsc.subcore_barrier()
    base = wid * EDGES_PER_W

    def chunk_body(i, carry):
        off = pl.multiple_of(base + i * CHUNK, 8)
        pltpu.sync_copy(src_hbm.at[pl.ds(off, CHUNK)], sidx_v)
        pltpu.sync_copy(dst_hbm.at[pl.ds(off, CHUNK)], didx_v)
        pltpu.sync_copy(ew_hbm.at[pl.ds(off, CHUNK)], ew_v)
        pltpu.async_copy(g_hbm.at[sidx_v], rows_v, sem).wait()

        def scale_body(q, c2):
            wv = ew_v[pl.ds(q * 16, 16)]
            for k in range(16):
                w = wv[k]
                row = q * 16 + k
                for j in range(D // 16):
                    sl = pl.ds(j * 16, 16)
                    rows_v[row, sl] = rows_v[row, sl] * w
            return c2

        lax.fori_loop(0, CHUNK // 16, scale_body, 0)
        pltpu.sync_copy(rows_v, acc_sp.at[didx_v], add=True)
        return carry

    lax.fori_loop(0, NCHUNK, chunk_body, 0)
    plsc.subcore_barrier()
    pltpu.sync_copy(acc_sp.at[pl.ds(row_lo, ROWS_PER_TILE)],
                    out_hbm.at[c, pl.ds(row_lo, ROWS_PER_TILE)])


def _mm1_body(x_ref, w_ref, degp_ref, g_ref, dinv_ref):
    deg = degp_ref[:, 0:1] + degp_ref[:, 1:2] + 1.0
    dinv = lax.rsqrt(deg)
    h = jnp.dot(x_ref[...], w_ref[...], preferred_element_type=jnp.float32,
                 precision=lax.Precision.HIGHEST)
    g_ref[...] = h * dinv
    dinv_ref[...] = dinv


def _mid_body(accp_ref, g_ref, dinv_ref, b_ref, w_ref, o_ref):
    a = accp_ref[0, :N] + accp_ref[1, :N] + g_ref[...]
    h = jnp.maximum(a * dinv_ref[...] + b_ref[...], 0.0)
    h2 = jnp.dot(h, w_ref[...], preferred_element_type=jnp.float32,
                 precision=lax.Precision.HIGHEST)
    o_ref[...] = h2 * dinv_ref[...]


def _final_body(accp_ref, g_ref, dinv_ref, b_ref, batch_ref, wh_ref, bh_ref,
                o_ref):
    a = accp_ref[0, :N] + accp_ref[1, :N] + g_ref[...]
    h = jnp.maximum(a * dinv_ref[...] + b_ref[...], 0.0)
    gids = lax.broadcasted_iota(jnp.int32, (G, N), 0)
    m = (batch_ref[...] == gids).astype(jnp.float32)
    sums = jnp.dot(m, h, preferred_element_type=jnp.float32,
                 precision=lax.Precision.HIGHEST)
    cnts = jnp.sum(m, axis=1, keepdims=True)
    pooled = sums / jnp.maximum(cnts, 1.0)
    o_ref[...] = (jnp.dot(pooled, wh_ref[...],
                          preferred_element_type=jnp.float32,
                 precision=lax.Precision.HIGHEST) + bh_ref[...])


_mm1 = pl.pallas_call(
    _mm1_body,
    out_shape=(jax.ShapeDtypeStruct((N, D), jnp.float32),
               jax.ShapeDtypeStruct((N, 1), jnp.float32)),
)

_mid = pl.pallas_call(
    _mid_body,
    out_shape=jax.ShapeDtypeStruct((N, D), jnp.float32),
)

_final = pl.pallas_call(
    _final_body,
    out_shape=jax.ShapeDtypeStruct((G, 1), jnp.float32),
)


def kernel(x, edge_index, edge_weight, batch, W1, b1, W2, b2, Wh, bh):
    src = edge_index[0]
    dst = edge_index[1]
    zeros_deg = jnp.zeros((NPAD,), jnp.float32)
    zeros_rows = jnp.zeros((NPAD, D), jnp.float32)

    degp = _deg_sc(dst, edge_weight, zeros_deg)
    degp_t = degp[:, :N].T

    g1, dinv = _mm1(x, W1, degp_t)
    acc1 = _agg_sc(g1, src, dst, edge_weight, zeros_rows)
    g2 = _mid(acc1, g1, dinv, b1.reshape(1, D), W2)
    acc2 = _agg_sc(g2, src, dst, edge_weight, zeros_rows)
    out = _final(acc2, g2, dinv, b2.reshape(1, D), batch.reshape(1, N),
                 Wh, bh.reshape(1, 1))
    return out


# speedup vs baseline: 14.8755x; 1.5070x over previous
import functools

import jax
import jax.numpy as jnp
from jax import lax
from jax.experimental import pallas as pl
from jax.experimental.pallas import tpu as pltpu
from jax.experimental.pallas import tpu_sc as plsc

N = 10000
E = 320000
D = 128
G = 64

NC = 2
NS = 16
NW = NC * NS
EDGES_PER_W = E // NW
CHUNK = 128
NCHUNK = 79
EPADW = NCHUNK * CHUNK
NPAD = 10240
ROWS_PER_TILE = NPAD // NS

_sc_mesh = plsc.VectorSubcoreMesh(core_axis_name="c", subcore_axis_name="s")


@functools.partial(
    pl.kernel,
    out_type=jax.ShapeDtypeStruct((NC, NPAD), jnp.float32),
    mesh=_sc_mesh,
    scratch_types=[
        pltpu.VMEM((NCHUNK, CHUNK), jnp.int32),
        pltpu.VMEM((NCHUNK, CHUNK), jnp.float32),
        pltpu.VMEM_SHARED((NPAD,), jnp.float32),
        pltpu.SemaphoreType.DMA,
    ],
)
def _deg_sc(dst_hbm, ew_hbm, zeros_hbm, out_hbm, dstb_v, ewb_v, acc_sp, sem):
    c = lax.axis_index("c")
    s = lax.axis_index("s")
    wid = s * NC + c
    tile_lo = s * ROWS_PER_TILE
    pltpu.sync_copy(zeros_hbm.at[pl.ds(tile_lo, ROWS_PER_TILE)],
                    acc_sp.at[pl.ds(tile_lo, ROWS_PER_TILE)])
    pltpu.sync_copy(dst_hbm.at[wid], dstb_v)
    pltpu.sync_copy(ew_hbm.at[wid], ewb_v)
    plsc.subcore_barrier()

    def chunk_body(i, carry):
        pltpu.sync_copy(ewb_v.at[i], acc_sp.at[dstb_v.at[i]], add=True)
        return carry

    lax.fori_loop(0, NCHUNK, chunk_body, 0)
    plsc.subcore_barrier()
    pltpu.sync_copy(acc_sp.at[pl.ds(tile_lo, ROWS_PER_TILE)],
                    out_hbm.at[c, pl.ds(tile_lo, ROWS_PER_TILE)])


@functools.partial(
    pl.kernel,
    out_type=jax.ShapeDtypeStruct((NC, NPAD, D), jnp.float32),
    mesh=_sc_mesh,
    scratch_types=[
        pltpu.VMEM((2, CHUNK), jnp.int32),
        pltpu.VMEM((2, CHUNK), jnp.int32),
        pltpu.VMEM((2, CHUNK), jnp.float32),
        pltpu.VMEM((2, CHUNK, D), jnp.float32),
        pltpu.VMEM_SHARED((NPAD, D), jnp.float32),
        pltpu.SemaphoreType.DMA,
        pltpu.SemaphoreType.DMA,
        pltpu.SemaphoreType.DMA,
        pltpu.SemaphoreType.DMA,
    ],
)
def _agg_sc(g_hbm, src_hbm, dst_hbm, ew_hbm, zrows_hbm, out_hbm,
            srcb_v, dstb_v, ewb_v, rows_v, acc_sp,
            isem0, isem1, gsem0, gsem1):
    c = lax.axis_index("c")
    s = lax.axis_index("s")
    wid = s * NC + c
    row_lo = s * ROWS_PER_TILE
    pltpu.sync_copy(zrows_hbm.at[pl.ds(row_lo, ROWS_PER_TILE)],
                    acc_sp.at[pl.ds(row_lo, ROWS_PER_TILE)])
    isems = (isem0, isem1)
    gsems = (gsem0, gsem1)

    def idx_fetch(i, b):
        sem = isems[b]
        pltpu.async_copy(src_hbm.at[wid, pl.ds(i, 1)], srcb_v.at[pl.ds(b, 1)], sem)
        pltpu.async_copy(dst_hbm.at[wid, pl.ds(i, 1)], dstb_v.at[pl.ds(b, 1)], sem)
        pltpu.async_copy(ew_hbm.at[wid, pl.ds(i, 1)], ewb_v.at[pl.ds(b, 1)], sem)

    def idx_wait(b):
        sem = isems[b]
        pltpu.make_async_copy(src_hbm.at[0, pl.ds(0, 1)], srcb_v.at[pl.ds(b, 1)], sem).wait()
        pltpu.make_async_copy(dst_hbm.at[0, pl.ds(0, 1)], dstb_v.at[pl.ds(b, 1)], sem).wait()
        pltpu.make_async_copy(ew_hbm.at[0, pl.ds(0, 1)], ewb_v.at[pl.ds(b, 1)], sem).wait()

    def gather(b):
        pltpu.async_copy(g_hbm.at[srcb_v.at[b]], rows_v.at[b], gsems[b])

    def gather_wait(b):
        pltpu.make_async_copy(g_hbm.at[srcb_v.at[b]], rows_v.at[b],
                              gsems[b]).wait()

    def scale(b):
        def scale_body(q, c2):
            wv = ewb_v[b, pl.ds(q * 16, 16)]
            for k in range(16):
                w = wv[k]
                row = q * 16 + k
                for j in range(D // 16):
                    sl = pl.ds(j * 16, 16)
                    rows_v[b, row, sl] = rows_v[b, row, sl] * w
            return c2
        lax.fori_loop(0, CHUNK // 16, scale_body, 0)

    def scatter(b):
        pltpu.sync_copy(rows_v.at[b], acc_sp.at[dstb_v.at[b]], add=True)

    idx_fetch(0, 0)
    idx_fetch(1, 1)
    idx_wait(0)
    gather(0)

    def step(i, b):
        other = 1 - b
        idx_wait(other)
        gather(other)
        gather_wait(b)
        scale(b)
        scatter(b)
        idx_fetch(i + 2, b)

    def pair_body(p, carry):
        step(2 * p, 0)
        step(2 * p + 1, 1)
        return carry

    lax.fori_loop(0, (NCHUNK - 3) // 2, pair_body, 0)
    idx_wait(1)
    gather(1)
    gather_wait(0)
    scale(0)
    scatter(0)
    idx_fetch(NCHUNK - 1, 0)
    idx_wait(0)
    gather(0)
    gather_wait(1)
    scale(1)
    scatter(1)
    gather_wait(0)
    scale(0)
    scatter(0)

    plsc.subcore_barrier()
    pltpu.sync_copy(acc_sp.at[pl.ds(row_lo, ROWS_PER_TILE)],
                    out_hbm.at[c, pl.ds(row_lo, ROWS_PER_TILE)])


def _mm1_body(x_ref, w_ref, degp_ref, g_ref, dinv_ref):
    deg = degp_ref[:, 0:1] + degp_ref[:, 1:2] + 1.0
    dinv = 1.0 / jnp.sqrt(deg)
    h = jnp.dot(x_ref[...], w_ref[...], preferred_element_type=jnp.float32,
                precision=lax.Precision.HIGHEST)
    g_ref[...] = h * dinv
    dinv_ref[...] = dinv


def _mid_body(accp_ref, g_ref, dinv_ref, b_ref, w_ref, o_ref):
    a = accp_ref[0, :N] + accp_ref[1, :N] + g_ref[...]
    h = jnp.maximum(a * dinv_ref[...] + b_ref[...], 0.0)
    h2 = jnp.dot(h, w_ref[...], preferred_element_type=jnp.float32,
                 precision=lax.Precision.HIGHEST)
    o_ref[...] = h2 * dinv_ref[...]


def _final_body(accp_ref, g_ref, dinv_ref, b_ref, batch_ref, wh_ref, bh_ref,
                o_ref):
    a = accp_ref[0, :N] + accp_ref[1, :N] + g_ref[...]
    h = jnp.maximum(a * dinv_ref[...] + b_ref[...], 0.0)
    gids = lax.broadcasted_iota(jnp.int32, (G, N), 0)
    m = (batch_ref[...] == gids).astype(jnp.float32)
    sums = jnp.dot(m, h, preferred_element_type=jnp.float32,
                   precision=lax.Precision.HIGHEST)
    cnts = jnp.sum(m, axis=1, keepdims=True)
    pooled = sums / jnp.maximum(cnts, 1.0)
    o_ref[...] = (jnp.dot(pooled, wh_ref[...], preferred_element_type=jnp.float32,
                          precision=lax.Precision.HIGHEST) + bh_ref[...])


_mm1 = pl.pallas_call(
    _mm1_body,
    out_shape=(jax.ShapeDtypeStruct((N, D), jnp.float32),
               jax.ShapeDtypeStruct((N, 1), jnp.float32)),
)

_mid = pl.pallas_call(
    _mid_body,
    out_shape=jax.ShapeDtypeStruct((N, D), jnp.float32),
)

_final = pl.pallas_call(
    _final_body,
    out_shape=jax.ShapeDtypeStruct((G, 1), jnp.float32),
)


def kernel(x, edge_index, edge_weight, batch, W1, b1, W2, b2, Wh, bh):
    pad = EPADW - EDGES_PER_W
    src3 = jnp.pad(edge_index[0].reshape(NW, EDGES_PER_W),
                   ((0, 0), (0, pad))).reshape(NW, NCHUNK, CHUNK)
    dst3 = jnp.pad(edge_index[1].reshape(NW, EDGES_PER_W),
                   ((0, 0), (0, pad)),
                   constant_values=NPAD - 1).reshape(NW, NCHUNK, CHUNK)
    ew3 = jnp.pad(edge_weight.reshape(NW, EDGES_PER_W),
                  ((0, 0), (0, pad))).reshape(NW, NCHUNK, CHUNK)
    zeros_deg = jnp.zeros((NPAD,), jnp.float32)
    zeros_rows = jnp.zeros((NPAD, D), jnp.float32)

    degp = _deg_sc(dst3, ew3, zeros_deg)
    degp_t = degp[:, :N].T

    g1, dinv = _mm1(x, W1, degp_t)
    acc1 = _agg_sc(g1, src3, dst3, ew3, zeros_rows)
    g2 = _mid(acc1, g1, dinv, b1.reshape(1, D), W2)
    acc2 = _agg_sc(g2, src3, dst3, ew3, zeros_rows)
    out = _final(acc2, g2, dinv, b2.reshape(1, D), batch.reshape(1, N),
                 Wh, bh.reshape(1, 1))
    return out
